# baseline (speedup 1.0000x reference)
import functools
import math

import jax
import jax.numpy as jnp
from jax.experimental import pallas as pl

_SUBSET_FRAC = 0.8

_dot = functools.partial(jnp.dot, preferred_element_type=jnp.float32)


def _split(a):
    hi = a.astype(jnp.bfloat16)
    lo = (a - hi.astype(jnp.float32)).astype(jnp.bfloat16)
    return hi, lo


def _dot3(ah, al, bh, bl):
    return _dot(ah, bh) + (_dot(al, bh) + _dot(ah, bl))


def _routing_body(x_ref, bm_ref, sg_ref, o_ref, *, k, out_caps, grp):
    bm = bm_ref[...]
    bmh = bm.astype(jnp.bfloat16)
    sgh = sg_ref[...].astype(jnp.bfloat16)
    oc = out_caps

    xhs, xls, n2ts = [], [], []
    for g in range(grp):
        x = x_ref[g]
        xh, xl = _split(x)
        sq = x * x
        sh, sl = _split(sq)
        n2 = _dot(sh, sgh) + _dot(sl, sgh)
        n2ts.append(n2.T)
        xhs.append(xh); xls.append(xl)

    n2t = jnp.concatenate(n2ts, axis=0)
    nt = jnp.sqrt(n2t)
    t1 = jnp.sum(nt, axis=1, keepdims=True)

    xvts, vsqs = [], []
    for g in range(grp):
        ntg = nt[g * oc:(g + 1) * oc]
        nth, ntl = _split(ntg)
        g1 = _dot3(nth, ntl, xhs[g], xls[g])
        s1 = jnp.sum(g1 * bm, axis=0, keepdims=True)
        t1e = jnp.sum(t1[g * oc:(g + 1) * oc] * bm, axis=0, keepdims=True)
        v1 = s1 / t1e
        vdt = (v1 * bm).T
        vh, vl = _split(vdt)
        xv = _dot3(xhs[g], xls[g], vh, vl)
        xvts.append(xv.T)
        vsqs.append(jnp.sum((v1 * v1) * bm, axis=1, keepdims=True))

    xvt = jnp.concatenate(xvts, axis=0)
    vsq = jnp.concatenate(vsqs, axis=0)
    l2t = (n2t - 2.0 * xvt) + vsq

    bits = jax.lax.bitcast_convert_type(l2t, jnp.int32)

    lo = jnp.zeros((grp * oc, 1), jnp.int32)
    hi = jnp.full((grp * oc, 1), 0x7F800000, jnp.int32)
    for _ in range(31):
        mid = lo + ((hi - lo) >> 1)
        cnt = jnp.sum((bits <= mid).astype(jnp.int32), axis=1, keepdims=True)
        take = cnt >= k
        lo = jnp.where(take, lo, mid + 1)
        hi = jnp.where(take, mid, hi)

    nmt = jnp.where(bits <= lo, nt, 0.0)
    t2 = jnp.sum(nmt, axis=1, keepdims=True)
    for g in range(grp):
        nmh, nml = _split(nmt[g * oc:(g + 1) * oc])
        g2 = _dot3(nmh, nml, xhs[g], xls[g])
        s2 = jnp.sum(g2 * bm, axis=0, keepdims=True)
        t2e = jnp.sum(t2[g * oc:(g + 1) * oc] * bm, axis=0, keepdims=True)
        o_ref[g] = s2 / t2e


def kernel(u_predict):
    b, in_caps, out_caps, dim = u_predict.shape
    od = out_caps * dim
    k = math.ceil(_SUBSET_FRAC * in_caps)

    x = u_predict.reshape(b, in_caps, od)
    col = jnp.arange(od, dtype=jnp.int32) // dim
    bm = (jnp.arange(out_caps, dtype=jnp.int32)[:, None] == col[None, :])
    bm = bm.astype(jnp.float32)

    grp = 2 if b % 2 == 0 else 1
    out = pl.pallas_call(
        functools.partial(_routing_body, k=k, out_caps=out_caps, grp=grp),
        grid=(b // grp,),
        in_specs=[
            pl.BlockSpec((grp, in_caps, od), lambda i: (i, 0, 0)),
            pl.BlockSpec((out_caps, od), lambda i: (0, 0)),
            pl.BlockSpec((od, out_caps), lambda i: (0, 0)),
        ],
        out_specs=pl.BlockSpec((grp, 1, od), lambda i: (i, 0, 0)),
        out_shape=jax.ShapeDtypeStruct((b, 1, od), jnp.float32),
    )(x, bm, bm.T)
    return out.reshape(b, out_caps, dim)


# speedup vs baseline: 1.2963x; 1.2963x over previous
import functools
import math

import jax
import jax.numpy as jnp
from jax.experimental import pallas as pl

_SUBSET_FRAC = 0.8

_dot = functools.partial(jnp.dot, preferred_element_type=jnp.float32)


def _split(a):
    hi = a.astype(jnp.bfloat16)
    lo = (a - hi.astype(jnp.float32)).astype(jnp.bfloat16)
    return hi, lo


def _dot3(ah, al, bh, bl):
    return _dot(ah, bh) + (_dot(al, bh) + _dot(ah, bl))


def _routing_body(x_ref, bm_ref, o_ref, *, k, out_caps, grp):
    bm = bm_ref[...]
    bmh = bm.astype(jnp.bfloat16)
    oc = out_caps

    xhs, xls, xths, xtls, n2ts = [], [], [], [], []
    for g in range(grp):
        x = x_ref[g]
        xt = x.T
        xh, xl = _split(x)
        xth, xtl = _split(xt)
        sqt = xt * xt
        sh, sl = _split(sqt)
        n2ts.append(_dot(bmh, sh) + _dot(bmh, sl))
        xhs.append(xh); xls.append(xl)
        xths.append(xth); xtls.append(xtl)

    n2t = jnp.concatenate(n2ts, axis=0)
    nt = jnp.sqrt(n2t)
    t1 = jnp.sum(nt, axis=1, keepdims=True)

    xvts, vsqs = [], []
    for g in range(grp):
        ntg = nt[g * oc:(g + 1) * oc]
        nth, ntl = _split(ntg)
        g1 = _dot3(nth, ntl, xhs[g], xls[g])
        s1 = jnp.sum(g1 * bm, axis=0, keepdims=True)
        t1e = jnp.sum(t1[g * oc:(g + 1) * oc] * bm, axis=0, keepdims=True)
        v1 = s1 / t1e
        vdt = v1 * bm
        vh, vl = _split(vdt)
        xvts.append(_dot3(vh, vl, xths[g], xtls[g]))
        vsqs.append(jnp.sum((v1 * v1) * bm, axis=1, keepdims=True))

    xvt = jnp.concatenate(xvts, axis=0)
    vsq = jnp.concatenate(vsqs, axis=0)
    l2t = (n2t - 2.0 * xvt) + vsq

    bits = jax.lax.bitcast_convert_type(l2t, jnp.int32)

    lo = jnp.zeros((grp * oc, 1), jnp.int32)
    hi = jnp.full((grp * oc, 1), 0x7F800000, jnp.int32)
    for _ in range(31):
        mid = lo + ((hi - lo) >> 1)
        cnt = jnp.sum((bits <= mid).astype(jnp.int32), axis=1, keepdims=True)
        take = cnt >= k
        lo = jnp.where(take, lo, mid + 1)
        hi = jnp.where(take, mid, hi)

    nmt = jnp.where(bits <= lo, nt, 0.0)
    t2 = jnp.sum(nmt, axis=1, keepdims=True)
    for g in range(grp):
        nmh, nml = _split(nmt[g * oc:(g + 1) * oc])
        g2 = _dot3(nmh, nml, xhs[g], xls[g])
        s2 = jnp.sum(g2 * bm, axis=0, keepdims=True)
        t2e = jnp.sum(t2[g * oc:(g + 1) * oc] * bm, axis=0, keepdims=True)
        o_ref[g] = s2 / t2e


def kernel(u_predict):
    b, in_caps, out_caps, dim = u_predict.shape
    od = out_caps * dim
    k = math.ceil(_SUBSET_FRAC * in_caps)

    x = u_predict.reshape(b, in_caps, od)
    col = jnp.arange(od, dtype=jnp.int32) // dim
    bm = (jnp.arange(out_caps, dtype=jnp.int32)[:, None] == col[None, :])
    bm = bm.astype(jnp.float32)

    grp = 4 if b % 4 == 0 else (2 if b % 2 == 0 else 1)
    out = pl.pallas_call(
        functools.partial(_routing_body, k=k, out_caps=out_caps, grp=grp),
        grid=(b // grp,),
        in_specs=[
            pl.BlockSpec((grp, in_caps, od), lambda i: (i, 0, 0)),
            pl.BlockSpec((out_caps, od), lambda i: (0, 0)),
        ],
        out_specs=pl.BlockSpec((grp, 1, od), lambda i: (i, 0, 0)),
        out_shape=jax.ShapeDtypeStruct((b, 1, od), jnp.float32),
    )(x, bm)
    return out.reshape(b, out_caps, dim)
